# baseline (speedup 1.0000x reference)
import functools

import jax
import jax.numpy as jnp
from jax import lax
from jax.experimental import pallas as pl
from jax.experimental.pallas import tpu as pltpu
from jax.experimental.pallas import tpu_sc as plsc

_L = 128
_SUB = 256
_NJ = _SUB // _L


def _make_gather(B, D, S, NB, n_workers):
    subs_w = B // _SUB // n_workers
    bblk = NB // _SUB
    mesh = plsc.VectorSubcoreMesh(core_axis_name="c", subcore_axis_name="s")
    nc = mesh.num_cores

    @functools.partial(
        pl.kernel,
        out_type=jax.ShapeDtypeStruct((S, D // 8, NB // _L, 8, _L),
                                      jnp.float32),
        mesh=mesh,
        scratch_types=[
            pltpu.VMEM((subs_w * _SUB,), jnp.int32),
            pltpu.VMEM((2, _NJ, _L, D), jnp.float32),
            pltpu.VMEM((2, _NJ, D, _L), jnp.float32),
            pltpu.SemaphoreType.DMA,
            pltpu.SemaphoreType.DMA,
            pltpu.SemaphoreType.DMA,
            pltpu.SemaphoreType.DMA,
        ],
        compiler_params=pltpu.CompilerParams(use_tc_tiling_on_sc=False,
                                             needs_layout_passes=False),
    )
    def k(idx_hbm, table_hbm, out_hbm, idx_v, rows_v, blk_v, g0, g1, w0, w1):
        wid = lax.axis_index("s") * nc + lax.axis_index("c")
        gsems = (g0, g1)
        wsems = (w0, w1)

        pltpu.sync_copy(idx_hbm.at[pl.ds(wid * subs_w * _SUB, subs_w * _SUB)],
                        idx_v)

        def fire_gather(i, slot):
            for j in range(_NJ):
                pltpu.async_copy(
                    table_hbm.at[idx_v.at[pl.ds(i * _SUB + j * _L, _L)]],
                    rows_v.at[slot, j],
                    gsems[slot],
                )

        def drain_gather(slot):
            for j in range(_NJ):
                pltpu.make_async_copy(
                    table_hbm.at[idx_v.at[pl.ds(j * _L, _L)]],
                    rows_v.at[slot, j],
                    gsems[slot],
                ).wait()

        def transpose(slot):
            lanes = lax.iota(jnp.int32, 16)
            sl = jnp.full((16,), slot, jnp.int32)
            jvs = [jnp.full((16,), j, jnp.int32) for j in range(_NJ)]
            lvs = [lanes + l16 * 16 for l16 in range(_L // 16)]

            def cbody(c, carry):
                cv = jnp.full((16,), c, jnp.int32)
                for j in range(_NJ):
                    for l16 in range(_L // 16):
                        vec = plsc.load_gather(
                            rows_v, [sl, jvs[j], lvs[l16], cv])
                        plsc.store_scatter(
                            blk_v, [sl, jvs[j], cv, lvs[l16]], vec)
                return carry

            lax.fori_loop(0, D, cbody, 0)

        def fire_writes(i, slot):
            u = wid * subs_w + i
            s = u // bblk
            bq = u - s * bblk
            for g in range(D // 8):
                for jb in range(_NJ):
                    pltpu.async_copy(
                        blk_v.at[slot, jb, pl.ds(g * 8, 8), :],
                        out_hbm.at[s, g, bq * _NJ + jb],
                        wsems[slot],
                    )

        def drain_writes(slot):
            for g in range(D // 8):
                for jb in range(_NJ):
                    pltpu.make_async_copy(
                        blk_v.at[slot, jb, pl.ds(g * 8, 8), :],
                        out_hbm.at[0, g, 0],
                        wsems[slot],
                    ).wait()

        fire_gather(0, 0)
        fire_gather(1, 1)

        def outer(o, carry):
            for s2 in range(2):
                i = o * 2 + s2
                drain_gather(s2)

                @pl.when(i >= 2)
                def _():
                    drain_writes(s2)

                transpose(s2)
                fire_writes(i, s2)

                @pl.when(i + 2 < subs_w)
                def _():
                    fire_gather(i + 2, s2)

            return carry

        lax.fori_loop(0, subs_w // 2, outer, 0)
        drain_writes(0)
        drain_writes(1)

    return k


def kernel(tokens, table):
    nb, s = tokens.shape
    _, d = table.shape
    idx = tokens.T.reshape(-1).astype(jnp.int32)
    out5 = _make_gather(idx.shape[0], d, s, nb, 32)(idx, table)
    return out5.transpose(2, 4, 0, 1, 3).reshape(nb, s, d)


# speedup vs baseline: 1.4857x; 1.4857x over previous
import functools

import jax
import jax.numpy as jnp
from jax import lax
from jax.experimental import pallas as pl
from jax.experimental.pallas import tpu as pltpu
from jax.experimental.pallas import tpu_sc as plsc

_L = 128
_CHUNK = 512
_NBUF = 2


def _make_gather(B, D, n_workers):
    b_per_w = B // n_workers
    n_chunks = b_per_w // _CHUNK
    assert n_chunks % _NBUF == 0
    mesh = plsc.VectorSubcoreMesh(core_axis_name="c", subcore_axis_name="s")
    nc = mesh.num_cores

    @functools.partial(
        pl.kernel,
        out_type=jax.ShapeDtypeStruct((B, D), jnp.float32),
        mesh=mesh,
        scratch_types=[
            pltpu.VMEM((n_chunks, _CHUNK // _L, _L), jnp.int32),
            pltpu.VMEM((_NBUF, _CHUNK, D), jnp.float32),
            pltpu.SemaphoreType.DMA,
            pltpu.SemaphoreType.DMA,
        ],
        compiler_params=pltpu.CompilerParams(use_tc_tiling_on_sc=False),
    )
    def k(idx_hbm, table_hbm, out_hbm, idx_v, rows_v, sem0, sem1):
        wid = lax.axis_index("s") * nc + lax.axis_index("c")
        c0 = wid * n_chunks
        sems = (sem0, sem1)

        pltpu.sync_copy(idx_hbm.at[pl.ds(c0, n_chunks)], idx_v)

        def fire(g, slot):
            for j in range(_CHUNK // _L):
                pltpu.async_copy(
                    table_hbm.at[idx_v.at[g, j]],
                    rows_v.at[slot, pl.ds(j * _L, _L), :],
                    sems[slot],
                )

        def drain(g, slot):
            for j in range(_CHUNK // _L):
                pltpu.make_async_copy(
                    table_hbm.at[idx_v.at[g, j]],
                    rows_v.at[slot, pl.ds(j * _L, _L), :],
                    sems[slot],
                ).wait()

        for b in range(_NBUF):
            fire(b, b)

        def outer(i, carry):
            for s in range(_NBUF):
                g = i * _NBUF + s
                drain(g, s)
                pltpu.sync_copy(
                    rows_v.at[s],
                    out_hbm.at[pl.ds((c0 + g) * _CHUNK, _CHUNK), :],
                )

                @pl.when(g + _NBUF < n_chunks)
                def _():
                    fire(g + _NBUF, s)

            return carry

        lax.fori_loop(0, n_chunks // _NBUF, outer, 0)

    return k


def kernel(tokens, table):
    b0, s = tokens.shape
    _, d = table.shape
    idx = tokens.reshape(-1).astype(jnp.int32)
    b = idx.shape[0]
    idx2 = idx.reshape(b // _CHUNK, _CHUNK // _L, _L)
    out = _make_gather(b, d, 32)(idx2, table)
    return out.reshape(b0, s, d)
